# baseline (speedup 1.0000x reference)
import jax
import jax.numpy as jnp
from jax.experimental import pallas as pl
from jax.experimental.pallas import tpu as pltpu

_RB = 128
_NCH = 4


def _router_kernel(x_ref, gw_ref, logits_ref, routT_ref, slotT_ref, posT_ref):
    x = x_ref[...]
    gw = gw_ref[...]
    logits = jax.lax.dot_general(x, gw, (((1,), (1,)), ((), ())),
                                 preferred_element_type=jnp.float32)
    logits_ref[...] = logits
    E, T = gw.shape[0], x.shape[0]
    lt = jax.lax.dot_general(gw, x, (((1,), (1,)), ((), ())),
                             preferred_element_type=jnp.float32)
    m = jnp.max(lt, axis=0, keepdims=True)
    ex = jnp.exp(lt - m)
    p = ex / jnp.sum(ex, axis=0, keepdims=True)
    eio = jax.lax.broadcasted_iota(jnp.int32, (E, T), 0)
    m1 = jnp.max(p, axis=0, keepdims=True)
    i1 = jnp.min(jnp.where(p == m1, eio, E), axis=0, keepdims=True)
    sel1 = eio == i1
    pm = jnp.where(sel1, -1.0, p)
    m2 = jnp.max(pm, axis=0, keepdims=True)
    i2 = jnp.min(jnp.where(pm == m2, eio, E), axis=0, keepdims=True)
    sel2 = eio == i2
    s = m1 + m2
    routT_ref[...] = jnp.where(sel1, m1 / s, 0.0) + jnp.where(sel2, m2 / s, 0.0)
    chosen = sel1 | sel2
    mf = chosen.astype(jnp.float32)
    tio_r = jax.lax.broadcasted_iota(jnp.int32, (T, T), 0)
    tio_c = jax.lax.broadcasted_iota(jnp.int32, (T, T), 1)
    tri = (tio_r <= tio_c).astype(jnp.float32)
    posT = jax.lax.dot_general(mf, tri, (((1,), (0,)), ((), ())),
                               preferred_element_type=jnp.float32)
    posT_i = posT.astype(jnp.int32)
    slotT_ref[...] = jnp.where(chosen, posT_i - 1, -1)
    posT_ref[...] = posT_i


def _issue_weight_dmas(e, slot, wg_hbm, wu_hbm, wd_hbm, bufs, sems):
    rows = wg_hbm.shape[1]
    ch = rows // _NCH
    for src, dst in zip((wg_hbm, wu_hbm, wd_hbm), bufs):
        for c in range(_NCH):
            pltpu.make_async_copy(
                src.at[e, pl.ds(c * ch, ch), :],
                dst.at[slot, pl.ds(c * ch, ch), :],
                sems.at[slot],
            ).start()


def _wait_weight_dmas(e, slot, wg_hbm, wu_hbm, wd_hbm, bufs, sems):
    for src, dst in zip((wg_hbm, wu_hbm, wd_hbm), bufs):
        pltpu.make_async_copy(src.at[e], dst.at[slot], sems.at[slot]).wait()


def _expert_kernel(counts_ref, slot_ref, rout_ref, x_ref, wg_hbm, wu_hbm,
                   wd_hbm, out_ref, wg_buf, wu_buf, wd_buf, sems):
    e = pl.program_id(0)
    E = pl.num_programs(0)
    bufs = (wg_buf, wu_buf, wd_buf)
    slot = jax.lax.rem(e, 2)

    @pl.when(e == 0)
    def _prologue():
        out_ref[...] = jnp.zeros_like(out_ref)
        _issue_weight_dmas(0, 0, wg_hbm, wu_hbm, wd_hbm, bufs, sems)

    @pl.when(e + 1 < E)
    def _prefetch_next():
        _issue_weight_dmas(e + 1, jax.lax.rem(e + 1, 2), wg_hbm, wu_hbm,
                           wd_hbm, bufs, sems)

    _wait_weight_dmas(e, slot, wg_hbm, wu_hbm, wd_hbm, bufs, sems)

    cnt = counts_ref[e]
    nblk = (cnt + _RB - 1) // _RB
    slot_row = slot_ref[...]
    rout_row = rout_ref[...].astype(jnp.bfloat16)
    x = x_ref[...]
    wg = wg_buf[slot].astype(jnp.bfloat16)
    wu = wu_buf[slot].astype(jnp.bfloat16)
    wd = wd_buf[slot].astype(jnp.bfloat16)
    T = x.shape[0]

    def body(b, carry):
        row_ids = b * _RB + jax.lax.broadcasted_iota(jnp.int32, (_RB, T), 0)
        onehot = (slot_row == row_ids).astype(jnp.bfloat16)
        xg = jax.lax.dot_general(onehot, x, (((1,), (0,)), ((), ())),
                                 preferred_element_type=jnp.float32
                                 ).astype(jnp.bfloat16)
        g = jax.lax.dot_general(xg, wg, (((1,), (1,)), ((), ())),
                                preferred_element_type=jnp.float32)
        u = jax.lax.dot_general(xg, wu, (((1,), (1,)), ((), ())),
                                preferred_element_type=jnp.float32)
        h = (g * jax.nn.sigmoid(g) * u).astype(jnp.bfloat16)
        dn = jax.lax.dot_general(h, wd, (((1,), (1,)), ((), ())),
                                 preferred_element_type=jnp.float32
                                 ).astype(jnp.bfloat16)
        ow = onehot * rout_row
        out_ref[...] += jax.lax.dot_general(ow, dn, (((0,), (0,)), ((), ())),
                                            preferred_element_type=jnp.float32)
        return carry

    jax.lax.fori_loop(0, nblk, body, 0)


def kernel(hidden_states, gate_weight, w_gate, w_up, w_down):
    T, H = hidden_states.shape
    E = gate_weight.shape[0]
    I = w_gate.shape[1]

    logits, routT, slotT, posT = pl.pallas_call(
        _router_kernel,
        out_shape=[
            jax.ShapeDtypeStruct((T, E), jnp.float32),
            jax.ShapeDtypeStruct((E, T), jnp.float32),
            jax.ShapeDtypeStruct((E, T), jnp.int32),
            jax.ShapeDtypeStruct((E, T), jnp.int32),
        ],
    )(hidden_states, gate_weight)

    counts = posT[:, T - 1]
    slotT3 = slotT.reshape(E, 1, T)
    routT3 = routT.reshape(E, 1, T)

    grid_spec = pltpu.PrefetchScalarGridSpec(
        num_scalar_prefetch=1,
        grid=(E,),
        in_specs=[
            pl.BlockSpec((None, 1, T), lambda e, c: (e, 0, 0)),
            pl.BlockSpec((None, 1, T), lambda e, c: (e, 0, 0)),
            pl.BlockSpec((T, H), lambda e, c: (0, 0)),
            pl.BlockSpec(memory_space=pl.ANY),
            pl.BlockSpec(memory_space=pl.ANY),
            pl.BlockSpec(memory_space=pl.ANY),
        ],
        out_specs=pl.BlockSpec((T, H), lambda e, c: (0, 0)),
        scratch_shapes=[
            pltpu.VMEM((2, I, H), jnp.float32),
            pltpu.VMEM((2, I, H), jnp.float32),
            pltpu.VMEM((2, H, I), jnp.float32),
            pltpu.SemaphoreType.DMA((2,)),
        ],
    )
    out = pl.pallas_call(
        _expert_kernel,
        grid_spec=grid_spec,
        out_shape=jax.ShapeDtypeStruct((T, H), jnp.float32),
    )(counts, slotT3, routT3, hidden_states.astype(jnp.bfloat16),
      w_gate, w_up, w_down)
    return out, logits


# speedup vs baseline: 1.0900x; 1.0900x over previous
import jax
import jax.numpy as jnp
from jax.experimental import pallas as pl
from jax.experimental.pallas import tpu as pltpu

_RB = 128
_NCH = 4


def _router_kernel(x_ref, gw_ref, logits_ref, routT_ref, slotT_ref, posT_ref):
    x = x_ref[...]
    gw = gw_ref[...]
    logits = jax.lax.dot_general(x, gw, (((1,), (1,)), ((), ())),
                                 preferred_element_type=jnp.float32)
    logits_ref[...] = logits
    E, T = gw.shape[0], x.shape[0]
    lt = jax.lax.dot_general(gw, x, (((1,), (1,)), ((), ())),
                             preferred_element_type=jnp.float32)
    m = jnp.max(lt, axis=0, keepdims=True)
    ex = jnp.exp(lt - m)
    p = ex / jnp.sum(ex, axis=0, keepdims=True)
    eio = jax.lax.broadcasted_iota(jnp.int32, (E, T), 0)
    m1 = jnp.max(p, axis=0, keepdims=True)
    i1 = jnp.min(jnp.where(p == m1, eio, E), axis=0, keepdims=True)
    sel1 = eio == i1
    pm = jnp.where(sel1, -1.0, p)
    m2 = jnp.max(pm, axis=0, keepdims=True)
    i2 = jnp.min(jnp.where(pm == m2, eio, E), axis=0, keepdims=True)
    sel2 = eio == i2
    s = m1 + m2
    routT_ref[...] = jnp.where(sel1, m1 / s, 0.0) + jnp.where(sel2, m2 / s, 0.0)
    chosen = sel1 | sel2
    mf = chosen.astype(jnp.float32)
    tio_r = jax.lax.broadcasted_iota(jnp.int32, (T, T), 0)
    tio_c = jax.lax.broadcasted_iota(jnp.int32, (T, T), 1)
    tri = (tio_r <= tio_c).astype(jnp.float32)
    posT = jax.lax.dot_general(mf, tri, (((1,), (0,)), ((), ())),
                               preferred_element_type=jnp.float32)
    posT_i = posT.astype(jnp.int32)
    slotT_ref[...] = jnp.where(chosen, posT_i - 1, -1)
    posT_ref[...] = posT_i


def _issue_weight_dmas(e, slot, wg_hbm, wu_hbm, wd_hbm, bufs, sems):
    rows = wg_hbm.shape[1]
    ch = rows // _NCH
    for src, dst in zip((wg_hbm, wu_hbm, wd_hbm), bufs):
        for c in range(_NCH):
            pltpu.make_async_copy(
                src.at[e, pl.ds(c * ch, ch), :],
                dst.at[slot, pl.ds(c * ch, ch), :],
                sems.at[slot],
            ).start()


def _wait_weight_dmas(e, slot, wg_hbm, wu_hbm, wd_hbm, bufs, sems):
    for src, dst in zip((wg_hbm, wu_hbm, wd_hbm), bufs):
        pltpu.make_async_copy(src.at[e], dst.at[slot], sems.at[slot]).wait()


def _expert_kernel(counts_ref, slot_ref, rout_ref, x_ref, wg_hbm, wu_hbm,
                   wd_hbm, out_ref, wg_buf, wu_buf, wd_buf, sems):
    e = pl.program_id(0)
    E = pl.num_programs(0)
    bufs = (wg_buf, wu_buf, wd_buf)
    slot = jax.lax.rem(e, 2)

    @pl.when(e == 0)
    def _prologue():
        out_ref[...] = jnp.zeros_like(out_ref)
        _issue_weight_dmas(0, 0, wg_hbm, wu_hbm, wd_hbm, bufs, sems)

    @pl.when(e + 1 < E)
    def _prefetch_next():
        _issue_weight_dmas(e + 1, jax.lax.rem(e + 1, 2), wg_hbm, wu_hbm,
                           wd_hbm, bufs, sems)

    _wait_weight_dmas(e, slot, wg_hbm, wu_hbm, wd_hbm, bufs, sems)

    cnt = counts_ref[e]
    nblk = (cnt + _RB - 1) // _RB
    slot_row = slot_ref[...]
    rout_row = rout_ref[...].astype(jnp.bfloat16)
    x = x_ref[...]
    T = x.shape[0]

    def body(b, carry):
        wg = wg_buf[slot].astype(jnp.bfloat16)
        wu = wu_buf[slot].astype(jnp.bfloat16)
        wd = wd_buf[slot].astype(jnp.bfloat16)
        row_ids = b * _RB + jax.lax.broadcasted_iota(jnp.int32, (_RB, T), 0)
        onehot = (slot_row == row_ids).astype(jnp.bfloat16)
        xg = jax.lax.dot_general(onehot, x, (((1,), (0,)), ((), ())),
                                 preferred_element_type=jnp.float32
                                 ).astype(jnp.bfloat16)
        g = jax.lax.dot_general(xg, wg, (((1,), (1,)), ((), ())),
                                preferred_element_type=jnp.float32)
        u = jax.lax.dot_general(xg, wu, (((1,), (1,)), ((), ())),
                                preferred_element_type=jnp.float32)
        h = (g * jax.nn.sigmoid(g) * u).astype(jnp.bfloat16)
        dn = jax.lax.dot_general(h, wd, (((1,), (1,)), ((), ())),
                                 preferred_element_type=jnp.float32
                                 ).astype(jnp.bfloat16)
        ow = onehot * rout_row
        out_ref[...] += jax.lax.dot_general(ow, dn, (((0,), (0,)), ((), ())),
                                            preferred_element_type=jnp.float32)
        return carry

    jax.lax.fori_loop(0, nblk, body, 0)


def kernel(hidden_states, gate_weight, w_gate, w_up, w_down):
    T, H = hidden_states.shape
    E = gate_weight.shape[0]
    I = w_gate.shape[1]

    logits, routT, slotT, posT = pl.pallas_call(
        _router_kernel,
        out_shape=[
            jax.ShapeDtypeStruct((T, E), jnp.float32),
            jax.ShapeDtypeStruct((E, T), jnp.float32),
            jax.ShapeDtypeStruct((E, T), jnp.int32),
            jax.ShapeDtypeStruct((E, T), jnp.int32),
        ],
    )(hidden_states, gate_weight)

    counts = posT[:, T - 1]
    slotT3 = slotT.reshape(E, 1, T)
    routT3 = routT.reshape(E, 1, T)

    grid_spec = pltpu.PrefetchScalarGridSpec(
        num_scalar_prefetch=1,
        grid=(E,),
        in_specs=[
            pl.BlockSpec((None, 1, T), lambda e, c: (e, 0, 0)),
            pl.BlockSpec((None, 1, T), lambda e, c: (e, 0, 0)),
            pl.BlockSpec((T, H), lambda e, c: (0, 0)),
            pl.BlockSpec(memory_space=pl.ANY),
            pl.BlockSpec(memory_space=pl.ANY),
            pl.BlockSpec(memory_space=pl.ANY),
        ],
        out_specs=pl.BlockSpec((T, H), lambda e, c: (0, 0)),
        scratch_shapes=[
            pltpu.VMEM((2, I, H), jnp.float32),
            pltpu.VMEM((2, I, H), jnp.float32),
            pltpu.VMEM((2, H, I), jnp.float32),
            pltpu.SemaphoreType.DMA((2,)),
        ],
    )
    out = pl.pallas_call(
        _expert_kernel,
        grid_spec=grid_spec,
        out_shape=jax.ShapeDtypeStruct((T, H), jnp.float32),
    )(counts, slotT3, routT3, hidden_states.astype(jnp.bfloat16),
      w_gate, w_up, w_down)
    return out, logits


# speedup vs baseline: 1.1650x; 1.0688x over previous
import jax
import jax.numpy as jnp
from jax.experimental import pallas as pl
from jax.experimental.pallas import tpu as pltpu

_RB = 128
_NCH = 4
_NBUF = 3


def _router_kernel(x_ref, gw_ref, logits_ref, routT_ref, slotT_ref,
                   counts_ref, xb_ref):
    x = x_ref[...]
    gw = gw_ref[...]
    xb_ref[...] = x.astype(jnp.bfloat16)
    logits = jax.lax.dot_general(x, gw, (((1,), (1,)), ((), ())),
                                 preferred_element_type=jnp.float32)
    logits_ref[...] = logits
    E, T = gw.shape[0], x.shape[0]
    lt = jax.lax.dot_general(gw, x, (((1,), (1,)), ((), ())),
                             preferred_element_type=jnp.float32)
    m = jnp.max(lt, axis=0, keepdims=True)
    ex = jnp.exp(lt - m)
    p = ex / jnp.sum(ex, axis=0, keepdims=True)
    eio = jax.lax.broadcasted_iota(jnp.int32, (E, T), 0)
    m1 = jnp.max(p, axis=0, keepdims=True)
    i1 = jnp.min(jnp.where(p == m1, eio, E), axis=0, keepdims=True)
    sel1 = eio == i1
    pm = jnp.where(sel1, -1.0, p)
    m2 = jnp.max(pm, axis=0, keepdims=True)
    i2 = jnp.min(jnp.where(pm == m2, eio, E), axis=0, keepdims=True)
    sel2 = eio == i2
    s = m1 + m2
    routT_ref[...] = jnp.where(sel1, m1 / s, 0.0) + jnp.where(sel2, m2 / s, 0.0)
    chosen = sel1 | sel2
    mf = chosen.astype(jnp.float32)
    tio_r = jax.lax.broadcasted_iota(jnp.int32, (T, T), 0)
    tio_c = jax.lax.broadcasted_iota(jnp.int32, (T, T), 1)
    tri = (tio_r <= tio_c).astype(jnp.float32)
    posT = jax.lax.dot_general(mf, tri, (((1,), (0,)), ((), ())),
                               preferred_element_type=jnp.float32)
    posT_i = posT.astype(jnp.int32)
    slotT_ref[...] = jnp.where(chosen, posT_i - 1, -1)
    counts_ref[...] = jnp.max(posT_i, axis=1)


def _issue_weight_dmas(e, slot, wg_hbm, wu_hbm, wd_hbm, bufs, sems):
    rows = wg_hbm.shape[1]
    ch = rows // _NCH
    for src, dst in zip((wg_hbm, wu_hbm, wd_hbm), bufs):
        for c in range(_NCH):
            pltpu.make_async_copy(
                src.at[e, pl.ds(c * ch, ch), :],
                dst.at[slot, pl.ds(c * ch, ch), :],
                sems.at[slot],
            ).start()


def _wait_weight_dmas(e, slot, wg_hbm, wu_hbm, wd_hbm, bufs, sems):
    for src, dst in zip((wg_hbm, wu_hbm, wd_hbm), bufs):
        pltpu.make_async_copy(src.at[e], dst.at[slot], sems.at[slot]).wait()


def _expert_kernel(counts_ref, slot_ref, rout_ref, x_ref, wg_hbm, wu_hbm,
                   wd_hbm, out_ref, wg_buf, wu_buf, wd_buf, sems):
    e = pl.program_id(0)
    E = pl.num_programs(0)
    bufs = (wg_buf, wu_buf, wd_buf)
    slot = jax.lax.rem(e, _NBUF)

    @pl.when(e == 0)
    def _prologue():
        out_ref[...] = jnp.zeros_like(out_ref)
        _issue_weight_dmas(0, 0, wg_hbm, wu_hbm, wd_hbm, bufs, sems)
        _issue_weight_dmas(1, 1, wg_hbm, wu_hbm, wd_hbm, bufs, sems)

    @pl.when(e + 2 < E)
    def _prefetch_ahead():
        _issue_weight_dmas(e + 2, jax.lax.rem(e + 2, _NBUF), wg_hbm, wu_hbm,
                           wd_hbm, bufs, sems)

    _wait_weight_dmas(e, slot, wg_hbm, wu_hbm, wd_hbm, bufs, sems)

    cnt = counts_ref[e]
    nblk = (cnt + _RB - 1) // _RB
    slot_row = slot_ref[...]
    rout_row = rout_ref[...].astype(jnp.bfloat16)
    x = x_ref[...]
    T = x.shape[0]

    def body(b, carry):
        wg = wg_buf[slot].astype(jnp.bfloat16)
        wu = wu_buf[slot].astype(jnp.bfloat16)
        wd = wd_buf[slot].astype(jnp.bfloat16)
        row_ids = b * _RB + jax.lax.broadcasted_iota(jnp.int32, (_RB, T), 0)
        onehot = (slot_row == row_ids).astype(jnp.bfloat16)
        xg = jax.lax.dot_general(onehot, x, (((1,), (0,)), ((), ())),
                                 preferred_element_type=jnp.float32
                                 ).astype(jnp.bfloat16)
        g = jax.lax.dot_general(xg, wg, (((1,), (1,)), ((), ())),
                                preferred_element_type=jnp.float32)
        u = jax.lax.dot_general(xg, wu, (((1,), (1,)), ((), ())),
                                preferred_element_type=jnp.float32)
        h = (g * jax.nn.sigmoid(g) * u).astype(jnp.bfloat16)
        dn = jax.lax.dot_general(h, wd, (((1,), (1,)), ((), ())),
                                 preferred_element_type=jnp.float32
                                 ).astype(jnp.bfloat16)
        ow = onehot * rout_row
        out_ref[...] += jax.lax.dot_general(ow, dn, (((0,), (0,)), ((), ())),
                                            preferred_element_type=jnp.float32)
        return carry

    jax.lax.fori_loop(0, nblk, body, 0)


def kernel(hidden_states, gate_weight, w_gate, w_up, w_down):
    T, H = hidden_states.shape
    E = gate_weight.shape[0]
    I = w_gate.shape[1]

    logits, routT, slotT, counts, xb = pl.pallas_call(
        _router_kernel,
        out_shape=[
            jax.ShapeDtypeStruct((T, E), jnp.float32),
            jax.ShapeDtypeStruct((E, T), jnp.float32),
            jax.ShapeDtypeStruct((E, T), jnp.int32),
            jax.ShapeDtypeStruct((E,), jnp.int32),
            jax.ShapeDtypeStruct((T, H), jnp.bfloat16),
        ],
    )(hidden_states, gate_weight)

    slotT3 = slotT.reshape(E, 1, T)
    routT3 = routT.reshape(E, 1, T)

    grid_spec = pltpu.PrefetchScalarGridSpec(
        num_scalar_prefetch=1,
        grid=(E,),
        in_specs=[
            pl.BlockSpec((None, 1, T), lambda e, c: (e, 0, 0)),
            pl.BlockSpec((None, 1, T), lambda e, c: (e, 0, 0)),
            pl.BlockSpec((T, H), lambda e, c: (0, 0)),
            pl.BlockSpec(memory_space=pl.ANY),
            pl.BlockSpec(memory_space=pl.ANY),
            pl.BlockSpec(memory_space=pl.ANY),
        ],
        out_specs=pl.BlockSpec((T, H), lambda e, c: (0, 0)),
        scratch_shapes=[
            pltpu.VMEM((_NBUF, I, H), jnp.float32),
            pltpu.VMEM((_NBUF, I, H), jnp.float32),
            pltpu.VMEM((_NBUF, H, I), jnp.float32),
            pltpu.SemaphoreType.DMA((_NBUF,)),
        ],
    )
    out = pl.pallas_call(
        _expert_kernel,
        grid_spec=grid_spec,
        out_shape=jax.ShapeDtypeStruct((T, H), jnp.float32),
    )(counts, slotT3, routT3, xb, w_gate, w_up, w_down)
    return out, logits
